# Initial kernel scaffold; baseline (speedup 1.0000x reference)
import jax
import jax.numpy as jnp
from jax.experimental import pallas as pl


def kernel(x, l1_w, l1_b, l1_gamma, l1_beta, l1_mean, l1_var, l2_w, l2_b, l2_gamma, l2_beta, l2_mean, l2_var, fc_w, fc_b):
    raise NotImplementedError("write your pallas kernel here")


# speedup vs baseline: 36.1206x; 36.1206x over previous
import jax
import jax.numpy as jnp
from jax.experimental import pallas as pl
from jax.experimental.pallas import tpu as pltpu

_VMEM_LIMIT = 64 * 1024 * 1024


def _fused_body(x_ref, t1_ref, s1_ref, h1_ref, t2_ref, s2_ref, h2_ref,
                wfc_ref, bfc_ref, o_ref,
                xp1, x1c, xp2, x2c):
    B = x_ref.shape[0]
    xp1[...] = jnp.zeros_like(xp1)
    xp1[:, 0:7, 66:94] = x_ref[:, :, 0:28]
    xp1[:, 0:7, 98:126] = x_ref[:, :, 28:56]
    xp1[:, 1:8, 2:30] = x_ref[:, :, 56:84]
    xp1[:, 1:8, 34:62] = x_ref[:, :, 84:112]
    x1c[:, :, 0:128] = xp1[:, 0:7, :]
    x1c[:, :, 128:256] = xp1[:, 1:8, :]
    y1 = jnp.dot(x1c[...].reshape(B * 7, 256), t1_ref[...],
                 preferred_element_type=jnp.float32).reshape(B, 7, 1792)
    wq = jnp.maximum(y1[:, :, :896], y1[:, :, 896:])
    hq = jnp.maximum(wq[:, :, :448], wq[:, :, 448:])
    p1 = jnp.maximum(hq * s1_ref[0] + h1_ref[0], 0.0)
    xp2[...] = jnp.zeros_like(xp2)
    xp2[:, 1:8, 32:256] = p1[:, :, 0:224]
    xp2[:, 1:8, 320:544] = p1[:, :, 224:448]
    x2c[:, :, 0:576] = xp2[:, 0:7, :]
    x2c[:, :, 576:1152] = xp2[:, 1:8, :]
    x2c[:, :, 1152:1728] = xp2[:, 2:9, :]
    y2 = jnp.dot(x2c[...].reshape(B * 7, 1728), t2_ref[...],
                 preferred_element_type=jnp.float32).reshape(B, 7, 896)
    wq2 = jnp.maximum(y2[:, :, :448], y2[:, :, 448:])
    hq2 = jnp.maximum(wq2[:, :, :224], wq2[:, :, 224:])
    p2 = jnp.maximum(hq2 * s2_ref[0] + h2_ref[0], 0.0)
    o_ref[...] = (jnp.dot(p2.reshape(B, 1568), wfc_ref[...],
                          preferred_element_type=jnp.float32) + bfc_ref[...])


def kernel(x, l1_w, l1_b, l1_gamma, l1_beta, l1_mean, l1_var,
           l2_w, l2_b, l2_gamma, l2_beta, l2_mean, l2_var,
           fc_w, fc_b):
    N = x.shape[0]
    xq = x.reshape(N, 7, 112)
    eps = 1e-5

    sc1 = l1_gamma / jnp.sqrt(l1_var + eps)
    sh1 = l1_beta + (l1_b - l1_mean) * sc1
    sg1 = jnp.where(sc1 >= 0, 1.0, -1.0)
    sc2 = l2_gamma / jnp.sqrt(l2_var + eps)
    sh2 = l2_beta + (l2_b - l2_mean) * sc2
    sg2 = jnp.where(sc2 >= 0, 1.0, -1.0)

    ar = jnp.arange
    bh1 = (ar(8)[:, None, None, None] - 2 * ar(2)[:, None, None]
           - ar(2)[:, None] == ar(5)).astype(jnp.float32)
    bw1 = (ar(32)[:, None, None, None] - 2 * ar(14)[:, None, None]
           - ar(2)[:, None] == ar(5)).astype(jnp.float32)
    w1s = l1_w[:, 0] * sg1[:, None, None]
    t1 = jnp.einsum('ropd,xwqe,cde->rxqpowc', bh1, bw1, w1s) \
            .reshape(256, 1792)

    bh2 = (ar(6)[:, None, None] - ar(2)[:, None] == ar(5)) \
        .astype(jnp.float32)
    bw2 = (ar(18)[:, None, None, None] - 2 * ar(7)[:, None, None]
           - ar(2)[:, None] == ar(5)).astype(jnp.float32)
    w2s = l2_w * sg2[:, None, None, None]
    t2 = jnp.einsum('rpd,xwqe,cide->rxiqpwc', bh2, bw2, w2s) \
            .reshape(1728, 896)

    s1v = jnp.tile(jnp.abs(sc1), 28).reshape(1, 448)
    h1v = jnp.tile(sh1, 28).reshape(1, 448)
    s2v = jnp.tile(jnp.abs(sc2), 7).reshape(1, 224)
    h2v = jnp.tile(sh2, 7).reshape(1, 224)

    wfc = fc_w.reshape(10, 32, 7, 7).transpose(0, 2, 3, 1).reshape(10, 1568).T
    bfc = fc_b.reshape(1, 10)

    B = 64
    while N % B:
        B //= 2

    return pl.pallas_call(
        _fused_body,
        out_shape=jax.ShapeDtypeStruct((N, 10), jnp.float32),
        grid=(N // B,),
        in_specs=[
            pl.BlockSpec((B, 7, 112), lambda i: (i, 0, 0)),
            pl.BlockSpec((256, 1792), lambda i: (0, 0)),
            pl.BlockSpec((1, 448), lambda i: (0, 0)),
            pl.BlockSpec((1, 448), lambda i: (0, 0)),
            pl.BlockSpec((1728, 896), lambda i: (0, 0)),
            pl.BlockSpec((1, 224), lambda i: (0, 0)),
            pl.BlockSpec((1, 224), lambda i: (0, 0)),
            pl.BlockSpec((1568, 10), lambda i: (0, 0)),
            pl.BlockSpec((1, 10), lambda i: (0, 0)),
        ],
        out_specs=pl.BlockSpec((B, 10), lambda i: (i, 0)),
        scratch_shapes=[
            pltpu.VMEM((B, 8, 128), jnp.float32),
            pltpu.VMEM((B, 7, 256), jnp.float32),
            pltpu.VMEM((B, 9, 576), jnp.float32),
            pltpu.VMEM((B, 7, 1728), jnp.float32),
        ],
        compiler_params=pltpu.CompilerParams(
            dimension_semantics=("parallel",),
            vmem_limit_bytes=_VMEM_LIMIT),
    )(xq, t1, s1v, h1v, t2, s2v, h2v, wfc, bfc)
